# Initial kernel scaffold; baseline (speedup 1.0000x reference)
import functools
import random as _pyrandom

import numpy as np
import jax
import jax.numpy as jnp
from jax import lax
from jax.experimental import pallas as pl
from jax.experimental.pallas import tpu as pltpu
from jax.experimental.pallas import tpu_sc as plsc

_RF = 8
_N_POSITIVE = 2
_K = 1
_MARGIN = 12.0
_LANES = 16
_CH = 16


def _pair_ids(rng, y, x, H, W):
    positive_ids = []
    negative_ids = []
    ix_nw = 0
    iy_nw = 0
    ix_se = ix_nw + 1
    iy_se = iy_nw + 1
    for _x in range(ix_nw, ix_se + 1):
        for _y in range(iy_nw, iy_se + 1):
            if 0 <= _x <= W and 0 <= _y <= H:
                f = (_x // _RF, _y // _RF)
                if f not in positive_ids:
                    positive_ids.append((_x, _y))
    iys = rng.choices(list(range(0, H // _RF)), k=10)
    ixs = rng.choices(list(range(0, W // _RF)), k=10)
    for cx, cy in zip(ixs, iys):
        if (cx, cy) in positive_ids:
            continue
        negative_ids.append((cx, cy))
    if len(positive_ids) > _N_POSITIVE:
        positive_ids = sorted(
            positive_ids, key=lambda e: (e[1] - y) ** 2 + (e[0] - x) ** 2
        )[:_N_POSITIVE]
    if len(negative_ids) > _N_POSITIVE * _K:
        negative_ids = list(
            sorted(negative_ids, key=lambda e: (e[1] - y) ** 2 + (e[0] - x) ** 2)
        )[::-1][: _N_POSITIVE * _K]
    return positive_ids, negative_ids


@functools.lru_cache(maxsize=None)
def _build_tables(B, H, W, n_workers):
    rng = _pyrandom.Random(0)
    Hf, Wf = H // _RF, W // _RF
    max_n = _N_POSITIVE * _K
    bs, hs, ws = [], [], []
    n0s, n1s, m0s, m1s = [], [], [], []
    for b in range(B):
        for h in range(Hf):
            for w in range(Wf):
                p_ids, n_ids = _pair_ids(rng, h * _RF, w * _RF, H, W)
                if len(p_ids) == 0 or len(n_ids) == 0:
                    continue
                ny = [e[1] for e in n_ids]
                nx = [e[0] for e in n_ids]
                m = [1.0] * len(n_ids)
                while len(ny) < max_n:
                    ny.append(0)
                    nx.append(0)
                    m.append(0.0)
                bs.append(b)
                hs.append(h)
                ws.append(w)
                n0s.append(b * Hf * Wf + ny[0] * Wf + nx[0])
                n1s.append(b * Hf * Wf + ny[1] * Wf + nx[1])
                m0s.append(m[0])
                m1s.append(m[1])
    M = len(bs)
    bs = np.array(bs, np.int32)
    hs = np.array(hs, np.int32)
    ws = np.array(ws, np.int32)
    aidx = bs * (Hf * Wf) + hs * Wf + ws
    bbase = bs * (Hf * Wf)
    cnt = np.array(m0s, np.float32) + np.array(m1s, np.float32)
    scale = 1.0 / (cnt * (1e-6 + M))
    w0 = np.array(m0s, np.float32) * scale
    w1 = np.array(m1s, np.float32) * scale

    chunk = n_workers * _CH
    M_pad = ((M + chunk - 1) // chunk) * chunk
    pad = M_pad - M

    def _p(a, val=0):
        return np.pad(a, (0, pad), constant_values=val)

    per_w = M_pad // n_workers
    idx_slab = np.stack(
        [_p(aidx), _p(np.array(n0s, np.int32)), _p(np.array(n1s, np.int32)), _p(bbase)],
        axis=0,
    ).reshape(4, n_workers, per_w).transpose(1, 0, 2).copy()
    w_slab = np.stack([_p(w0, 0.0), _p(w1, 0.0)], axis=0).reshape(
        2, n_workers, per_w
    ).transpose(1, 0, 2).copy()
    return idx_slab, w_slab, _p(bs), _p(hs), _p(ws), M, M_pad


def _sc_kernel(n_workers, n_cores, per_w, C, Wf):
    n_chunks = per_w // _CH
    cl = C // _LANES

    def body(sk_hbm, ref_hbm, idx_hbm, w_hbm, g_hbm, out_hbm,
             idx_v, w_v, g_v, ab, pb, n0b, n1b,
             a_v, p_v, v0_v, v1_v, out_v, sem):
        wid = lax.axis_index("s") * n_cores + lax.axis_index("c")
        pltpu.sync_copy(idx_hbm.at[wid], idx_v)
        pltpu.sync_copy(w_hbm.at[wid], w_v)
        pltpu.sync_copy(g_hbm.at[wid], g_v)

        def chunk_body(g, tot):
            coff = pl.multiple_of(g * _CH, _CH)
            ab[...] = idx_v[0, pl.ds(coff, _CH)]
            n0b[...] = idx_v[1, pl.ds(coff, _CH)]
            n1b[...] = idx_v[2, pl.ds(coff, _CH)]
            bbv = idx_v[3, pl.ds(coff, _CH)]
            px = g_v[0, pl.ds(coff, _CH)].astype(jnp.int32)
            py = g_v[1, pl.ds(coff, _CH)].astype(jnp.int32)
            pb[...] = bbv + py * Wf + px
            cpa = pltpu.async_copy(ref_hbm.at[ab], a_v, sem)
            cpp = pltpu.async_copy(sk_hbm.at[pb], p_v, sem)
            cp0 = pltpu.async_copy(sk_hbm.at[n0b], v0_v, sem)
            cp1 = pltpu.async_copy(sk_hbm.at[n1b], v1_v, sem)
            cpa.wait()
            cpp.wait()
            cp0.wait()
            cp1.wait()

            for c in range(_CH):
                def dist_body(t, carry):
                    ap, an0, an1 = carry
                    off = pl.multiple_of(t * _LANES, _LANES)
                    av = a_v[c, pl.ds(off, _LANES)]
                    d = av - p_v[c, pl.ds(off, _LANES)]
                    ap = ap + d * d
                    d = av - v0_v[c, pl.ds(off, _LANES)]
                    an0 = an0 + d * d
                    d = av - v1_v[c, pl.ds(off, _LANES)]
                    an1 = an1 + d * d
                    return ap, an0, an1

                z = jnp.zeros((_LANES,), jnp.float32)
                ap, an0, an1 = lax.fori_loop(0, cl, dist_body, (z, z, z))
                dp = jnp.sum(ap)
                dn0 = jnp.sum(an0)
                dn1 = jnp.sum(an1)
                w0c = w_v[0, coff + c]
                w1c = w_v[1, coff + c]
                tot = tot + (
                    jnp.maximum(dp - dn0 + _MARGIN, 0.0) * w0c
                    + jnp.maximum(dp - dn1 + _MARGIN, 0.0) * w1c
                )
            return tot

        tot = lax.fori_loop(0, n_chunks, chunk_body, jnp.float32(0.0))
        lane0 = lax.iota(jnp.int32, (_LANES,)) == 0
        out_v[...] = jnp.where(
            lane0, jnp.full((_LANES,), tot), jnp.zeros((_LANES,), jnp.float32)
        )
        pltpu.sync_copy(out_v, out_hbm.at[wid])

    return pl.kernel(
        body,
        out_type=jax.ShapeDtypeStruct((n_workers, _LANES), jnp.float32),
        mesh=plsc.VectorSubcoreMesh(core_axis_name="c", subcore_axis_name="s"),
        scratch_types=[
            pltpu.VMEM((4, per_w), jnp.int32),
            pltpu.VMEM((2, per_w), jnp.float32),
            pltpu.VMEM((2, per_w), jnp.float32),
            pltpu.VMEM((_CH,), jnp.int32),
            pltpu.VMEM((_CH,), jnp.int32),
            pltpu.VMEM((_CH,), jnp.int32),
            pltpu.VMEM((_CH,), jnp.int32),
            pltpu.VMEM((_CH, C), jnp.float32),
            pltpu.VMEM((_CH, C), jnp.float32),
            pltpu.VMEM((_CH, C), jnp.float32),
            pltpu.VMEM((_CH, C), jnp.float32),
            pltpu.VMEM((_LANES,), jnp.float32),
            pltpu.SemaphoreType.DMA,
        ],
    )


def kernel(sketch_context_vectors, ref_context_vectors, G):
    B, H, W, _ = G.shape
    _, C, Hf, Wf = sketch_context_vectors.shape
    info = plsc.get_sparse_core_info()
    n_cores, n_subcores = info.num_cores, info.num_subcores
    n_workers = n_cores * n_subcores

    idx_slab, w_slab, bs_p, hs_p, ws_p, M, M_pad = _build_tables(
        int(B), int(H), int(W), n_workers
    )
    per_w = M_pad // n_workers

    sk_rows = jnp.transpose(sketch_context_vectors, (0, 2, 3, 1)).reshape(
        B * Hf * Wf, C
    )
    ref_rows = jnp.transpose(ref_context_vectors, (0, 2, 3, 1)).reshape(
        B * Hf * Wf, C
    )
    gxy = G[bs_p, hs_p * _RF, ws_p * _RF, :]
    g_slab = jnp.transpose(gxy.reshape(n_workers, per_w, 2), (0, 2, 1))

    fn = _sc_kernel(n_workers, n_cores, per_w, int(C), int(Wf))
    partials = fn(
        sk_rows,
        ref_rows,
        jnp.asarray(idx_slab),
        jnp.asarray(w_slab),
        g_slab,
    )
    return jnp.sum(partials)


# speedup vs baseline: 2.4129x; 2.4129x over previous
import functools
import random as _pyrandom

import numpy as np
import jax
import jax.numpy as jnp
from jax import lax
from jax.experimental import pallas as pl
from jax.experimental.pallas import tpu as pltpu
from jax.experimental.pallas import tpu_sc as plsc

_RF = 8
_N_POSITIVE = 2
_K = 1
_MARGIN = 12.0
_LANES = 16
_CH = 16


def _pair_ids(rng, y, x, H, W):
    positive_ids = []
    negative_ids = []
    ix_nw = 0
    iy_nw = 0
    ix_se = ix_nw + 1
    iy_se = iy_nw + 1
    for _x in range(ix_nw, ix_se + 1):
        for _y in range(iy_nw, iy_se + 1):
            if 0 <= _x <= W and 0 <= _y <= H:
                f = (_x // _RF, _y // _RF)
                if f not in positive_ids:
                    positive_ids.append((_x, _y))
    iys = rng.choices(list(range(0, H // _RF)), k=10)
    ixs = rng.choices(list(range(0, W // _RF)), k=10)
    for cx, cy in zip(ixs, iys):
        if (cx, cy) in positive_ids:
            continue
        negative_ids.append((cx, cy))
    if len(positive_ids) > _N_POSITIVE:
        positive_ids = sorted(
            positive_ids, key=lambda e: (e[1] - y) ** 2 + (e[0] - x) ** 2
        )[:_N_POSITIVE]
    if len(negative_ids) > _N_POSITIVE * _K:
        negative_ids = list(
            sorted(negative_ids, key=lambda e: (e[1] - y) ** 2 + (e[0] - x) ** 2)
        )[::-1][: _N_POSITIVE * _K]
    return positive_ids, negative_ids


@functools.lru_cache(maxsize=None)
def _build_tables(B, H, W, n_workers):
    rng = _pyrandom.Random(0)
    Hf, Wf = H // _RF, W // _RF
    max_n = _N_POSITIVE * _K
    bs, hs, ws = [], [], []
    n0s, n1s, m0s, m1s = [], [], [], []
    for b in range(B):
        for h in range(Hf):
            for w in range(Wf):
                p_ids, n_ids = _pair_ids(rng, h * _RF, w * _RF, H, W)
                if len(p_ids) == 0 or len(n_ids) == 0:
                    continue
                ny = [e[1] for e in n_ids]
                nx = [e[0] for e in n_ids]
                m = [1.0] * len(n_ids)
                while len(ny) < max_n:
                    ny.append(0)
                    nx.append(0)
                    m.append(0.0)
                bs.append(b)
                hs.append(h)
                ws.append(w)
                n0s.append(b * Hf * Wf + ny[0] * Wf + nx[0])
                n1s.append(b * Hf * Wf + ny[1] * Wf + nx[1])
                m0s.append(m[0])
                m1s.append(m[1])
    M = len(bs)
    bs = np.array(bs, np.int32)
    hs = np.array(hs, np.int32)
    ws = np.array(ws, np.int32)
    aidx = bs * (Hf * Wf) + hs * Wf + ws
    bbase = bs * (Hf * Wf)
    cnt = np.array(m0s, np.float32) + np.array(m1s, np.float32)
    scale = 1.0 / (cnt * (1e-6 + M))
    w0 = np.array(m0s, np.float32) * scale
    w1 = np.array(m1s, np.float32) * scale

    chunk = n_workers * _CH
    M_pad = ((M + chunk - 1) // chunk) * chunk
    pad = M_pad - M

    def _p(a, val=0):
        return np.pad(a, (0, pad), constant_values=val)

    per_w = M_pad // n_workers
    idx_slab = np.stack(
        [_p(aidx), _p(np.array(n0s, np.int32)), _p(np.array(n1s, np.int32)), _p(bbase)],
        axis=0,
    ).reshape(4, n_workers, per_w).transpose(1, 0, 2).copy()
    w_slab = np.stack([_p(w0, 0.0), _p(w1, 0.0)], axis=0).reshape(
        2, n_workers, per_w
    ).transpose(1, 0, 2).copy()
    return idx_slab, w_slab, _p(bs), _p(hs), _p(ws), M, M_pad


def _sc_kernel(n_workers, n_cores, per_w, C, Wf):
    n_chunks = per_w // _CH
    cl = C // _LANES

    def body(sk_hbm, ref_hbm, idx_hbm, w_hbm, g_hbm, out_hbm,
             idx_v, w_v, g_v, ab, pb, n0b, n1b,
             a_v, p_v, v0_v, v1_v, accp_v, accn0_v, accn1_v, out_v, sem):
        wid = lax.axis_index("s") * n_cores + lax.axis_index("c")
        pltpu.sync_copy(idx_hbm.at[wid], idx_v)
        pltpu.sync_copy(w_hbm.at[wid], w_v)
        pltpu.sync_copy(g_hbm.at[wid], g_v)
        rowi = lax.iota(jnp.int32, _LANES)

        def chunk_body(g, tot):
            coff = pl.multiple_of(g * _CH, _CH)
            ab[...] = idx_v[0, pl.ds(coff, _CH)]
            n0b[...] = idx_v[1, pl.ds(coff, _CH)]
            n1b[...] = idx_v[2, pl.ds(coff, _CH)]
            bbv = idx_v[3, pl.ds(coff, _CH)]
            px = g_v[0, pl.ds(coff, _CH)].astype(jnp.int32)
            py = g_v[1, pl.ds(coff, _CH)].astype(jnp.int32)
            pb[...] = bbv + py * Wf + px
            w0vec = w_v[0, pl.ds(coff, _CH)]
            w1vec = w_v[1, pl.ds(coff, _CH)]
            cpa = pltpu.async_copy(ref_hbm.at[ab], a_v, sem)
            cpp = pltpu.async_copy(sk_hbm.at[pb], p_v, sem)
            cp0 = pltpu.async_copy(sk_hbm.at[n0b], v0_v, sem)
            cp1 = pltpu.async_copy(sk_hbm.at[n1b], v1_v, sem)
            cpa.wait()
            cpp.wait()
            cp0.wait()
            cp1.wait()

            for c in range(_CH):
                def dist_body(t, carry):
                    ap, an0, an1 = carry
                    off = pl.multiple_of(t * _LANES, _LANES)
                    av = a_v[c, pl.ds(off, _LANES)]
                    d = av - p_v[c, pl.ds(off, _LANES)]
                    ap = ap + d * d
                    d = av - v0_v[c, pl.ds(off, _LANES)]
                    an0 = an0 + d * d
                    d = av - v1_v[c, pl.ds(off, _LANES)]
                    an1 = an1 + d * d
                    return ap, an0, an1

                z = jnp.zeros((_LANES,), jnp.float32)
                ap, an0, an1 = lax.fori_loop(0, cl, dist_body, (z, z, z))
                accp_v[c, ...] = ap
                accn0_v[c, ...] = an0
                accn1_v[c, ...] = an1

            zz = jnp.zeros((_LANES,), jnp.float32)
            dpv, dn0v, dn1v = zz, zz, zz
            for l in range(_LANES):
                coli = jnp.full((_LANES,), l, jnp.int32)
                dpv = dpv + plsc.load_gather(accp_v, [rowi, coli])
                dn0v = dn0v + plsc.load_gather(accn0_v, [rowi, coli])
                dn1v = dn1v + plsc.load_gather(accn1_v, [rowi, coli])
            lvec = (
                jnp.maximum(dpv - dn0v + _MARGIN, 0.0) * w0vec
                + jnp.maximum(dpv - dn1v + _MARGIN, 0.0) * w1vec
            )
            return tot + lvec

        tot = lax.fori_loop(
            0, n_chunks, chunk_body, jnp.zeros((_LANES,), jnp.float32)
        )
        out_v[...] = tot
        pltpu.sync_copy(out_v, out_hbm.at[wid])

    return pl.kernel(
        body,
        out_type=jax.ShapeDtypeStruct((n_workers, _LANES), jnp.float32),
        mesh=plsc.VectorSubcoreMesh(core_axis_name="c", subcore_axis_name="s"),
        compiler_params=pltpu.CompilerParams(needs_layout_passes=False),
        scratch_types=[
            pltpu.VMEM((4, per_w), jnp.int32),
            pltpu.VMEM((2, per_w), jnp.float32),
            pltpu.VMEM((2, per_w), jnp.float32),
            pltpu.VMEM((_CH,), jnp.int32),
            pltpu.VMEM((_CH,), jnp.int32),
            pltpu.VMEM((_CH,), jnp.int32),
            pltpu.VMEM((_CH,), jnp.int32),
            pltpu.VMEM((_CH, C), jnp.float32),
            pltpu.VMEM((_CH, C), jnp.float32),
            pltpu.VMEM((_CH, C), jnp.float32),
            pltpu.VMEM((_CH, C), jnp.float32),
            pltpu.VMEM((_CH, _LANES), jnp.float32),
            pltpu.VMEM((_CH, _LANES), jnp.float32),
            pltpu.VMEM((_CH, _LANES), jnp.float32),
            pltpu.VMEM((_LANES,), jnp.float32),
            pltpu.SemaphoreType.DMA,
        ],
    )


def kernel(sketch_context_vectors, ref_context_vectors, G):
    B, H, W, _ = G.shape
    _, C, Hf, Wf = sketch_context_vectors.shape
    info = plsc.get_sparse_core_info()
    n_cores, n_subcores = info.num_cores, info.num_subcores
    n_workers = n_cores * n_subcores

    idx_slab, w_slab, bs_p, hs_p, ws_p, M, M_pad = _build_tables(
        int(B), int(H), int(W), n_workers
    )
    per_w = M_pad // n_workers

    sk_rows = jnp.transpose(sketch_context_vectors, (0, 2, 3, 1)).reshape(
        B * Hf * Wf, C
    )
    ref_rows = jnp.transpose(ref_context_vectors, (0, 2, 3, 1)).reshape(
        B * Hf * Wf, C
    )
    gxy = G[bs_p, hs_p * _RF, ws_p * _RF, :]
    g_slab = jnp.transpose(gxy.reshape(n_workers, per_w, 2), (0, 2, 1))

    fn = _sc_kernel(n_workers, n_cores, per_w, int(C), int(Wf))
    partials = fn(
        sk_rows,
        ref_rows,
        jnp.asarray(idx_slab),
        jnp.asarray(w_slab),
        g_slab,
    )
    return jnp.sum(partials)


# speedup vs baseline: 2.4135x; 1.0003x over previous
import functools
import random as _pyrandom

import numpy as np
import jax
import jax.numpy as jnp
from jax import lax
from jax.experimental import pallas as pl
from jax.experimental.pallas import tpu as pltpu
from jax.experimental.pallas import tpu_sc as plsc

_RF = 8
_N_POSITIVE = 2
_K = 1
_MARGIN = 12.0
_LANES = 16
_CH = 16


def _pair_ids(rng, y, x, H, W):
    positive_ids = []
    negative_ids = []
    ix_nw = 0
    iy_nw = 0
    ix_se = ix_nw + 1
    iy_se = iy_nw + 1
    for _x in range(ix_nw, ix_se + 1):
        for _y in range(iy_nw, iy_se + 1):
            if 0 <= _x <= W and 0 <= _y <= H:
                f = (_x // _RF, _y // _RF)
                if f not in positive_ids:
                    positive_ids.append((_x, _y))
    iys = rng.choices(list(range(0, H // _RF)), k=10)
    ixs = rng.choices(list(range(0, W // _RF)), k=10)
    for cx, cy in zip(ixs, iys):
        if (cx, cy) in positive_ids:
            continue
        negative_ids.append((cx, cy))
    if len(positive_ids) > _N_POSITIVE:
        positive_ids = sorted(
            positive_ids, key=lambda e: (e[1] - y) ** 2 + (e[0] - x) ** 2
        )[:_N_POSITIVE]
    if len(negative_ids) > _N_POSITIVE * _K:
        negative_ids = list(
            sorted(negative_ids, key=lambda e: (e[1] - y) ** 2 + (e[0] - x) ** 2)
        )[::-1][: _N_POSITIVE * _K]
    return positive_ids, negative_ids


@functools.lru_cache(maxsize=None)
def _build_tables(B, H, W, n_workers):
    rng = _pyrandom.Random(0)
    Hf, Wf = H // _RF, W // _RF
    max_n = _N_POSITIVE * _K
    bs, hs, ws = [], [], []
    n0s, n1s, m0s, m1s = [], [], [], []
    for b in range(B):
        for h in range(Hf):
            for w in range(Wf):
                p_ids, n_ids = _pair_ids(rng, h * _RF, w * _RF, H, W)
                if len(p_ids) == 0 or len(n_ids) == 0:
                    continue
                ny = [e[1] for e in n_ids]
                nx = [e[0] for e in n_ids]
                m = [1.0] * len(n_ids)
                while len(ny) < max_n:
                    ny.append(0)
                    nx.append(0)
                    m.append(0.0)
                bs.append(b)
                hs.append(h)
                ws.append(w)
                n0s.append(b * Hf * Wf + ny[0] * Wf + nx[0])
                n1s.append(b * Hf * Wf + ny[1] * Wf + nx[1])
                m0s.append(m[0])
                m1s.append(m[1])
    M = len(bs)
    bs = np.array(bs, np.int32)
    hs = np.array(hs, np.int32)
    ws = np.array(ws, np.int32)
    aidx = bs * (Hf * Wf) + hs * Wf + ws
    bbase = bs * (Hf * Wf)
    cnt = np.array(m0s, np.float32) + np.array(m1s, np.float32)
    scale = 1.0 / (cnt * (1e-6 + M))
    w0 = np.array(m0s, np.float32) * scale
    w1 = np.array(m1s, np.float32) * scale

    chunk = n_workers * _CH
    M_pad = ((M + chunk - 1) // chunk) * chunk
    pad = M_pad - M

    def _p(a, val=0):
        return np.pad(a, (0, pad), constant_values=val)

    per_w = M_pad // n_workers
    idx_slab = np.stack(
        [_p(aidx), _p(np.array(n0s, np.int32)), _p(np.array(n1s, np.int32)), _p(bbase)],
        axis=0,
    ).reshape(4, n_workers, per_w).transpose(1, 0, 2).copy()
    w_slab = np.stack([_p(w0, 0.0), _p(w1, 0.0)], axis=0).reshape(
        2, n_workers, per_w
    ).transpose(1, 0, 2).copy()
    return idx_slab, w_slab, _p(bs), _p(hs), _p(ws), M, M_pad


def _sc_kernel(n_workers, n_cores, per_w, C, Wf):
    n_chunks = per_w // _CH
    assert n_chunks % 2 == 0
    n2 = n_chunks // 2
    cl = C // _LANES

    def body(sk_hbm, ref_hbm, idx_hbm, w_hbm, g_hbm, out_hbm,
             idx_v, w_v, g_v,
             ab0, pb0, n0b0, n1b0, ab1, pb1, n0b1, n1b1,
             a0_v, p0_v, v00_v, v10_v, a1_v, p1_v, v01_v, v11_v,
             accp_v, accn0_v, accn1_v, out_v, sem0, sem1):
        wid = lax.axis_index("s") * n_cores + lax.axis_index("c")
        pltpu.sync_copy(idx_hbm.at[wid], idx_v)
        pltpu.sync_copy(w_hbm.at[wid], w_v)
        pltpu.sync_copy(g_hbm.at[wid], g_v)
        rowi = lax.iota(jnp.int32, _LANES)

        sets = (
            (ab0, pb0, n0b0, n1b0, a0_v, p0_v, v00_v, v10_v, sem0),
            (ab1, pb1, n0b1, n1b1, a1_v, p1_v, v01_v, v11_v, sem1),
        )

        def stage_and_issue(coff, s):
            ab, pb, n0b, n1b, a_v, p_v, v0_v, v1_v, sem = s
            ab[...] = idx_v[0, pl.ds(coff, _CH)]
            n0b[...] = idx_v[1, pl.ds(coff, _CH)]
            n1b[...] = idx_v[2, pl.ds(coff, _CH)]
            bbv = idx_v[3, pl.ds(coff, _CH)]
            px = g_v[0, pl.ds(coff, _CH)].astype(jnp.int32)
            py = g_v[1, pl.ds(coff, _CH)].astype(jnp.int32)
            pb[...] = bbv + py * Wf + px
            pltpu.async_copy(ref_hbm.at[ab], a_v, sem)
            pltpu.async_copy(sk_hbm.at[pb], p_v, sem)
            pltpu.async_copy(sk_hbm.at[n0b], v0_v, sem)
            pltpu.async_copy(sk_hbm.at[n1b], v1_v, sem)

        def drain(s):
            ab, pb, n0b, n1b, a_v, p_v, v0_v, v1_v, sem = s
            pltpu.make_async_copy(ref_hbm.at[ab], a_v, sem).wait()
            pltpu.make_async_copy(sk_hbm.at[pb], p_v, sem).wait()
            pltpu.make_async_copy(sk_hbm.at[n0b], v0_v, sem).wait()
            pltpu.make_async_copy(sk_hbm.at[n1b], v1_v, sem).wait()

        def compute(coff, s):
            ab, pb, n0b, n1b, a_v, p_v, v0_v, v1_v, sem = s
            w0vec = w_v[0, pl.ds(coff, _CH)]
            w1vec = w_v[1, pl.ds(coff, _CH)]
            for c in range(_CH):
                def dist_body(t, carry):
                    ap, an0, an1 = carry
                    off = pl.multiple_of(t * _LANES, _LANES)
                    av = a_v[c, pl.ds(off, _LANES)]
                    d = av - p_v[c, pl.ds(off, _LANES)]
                    ap = ap + d * d
                    d = av - v0_v[c, pl.ds(off, _LANES)]
                    an0 = an0 + d * d
                    d = av - v1_v[c, pl.ds(off, _LANES)]
                    an1 = an1 + d * d
                    return ap, an0, an1

                z = jnp.zeros((_LANES,), jnp.float32)
                ap, an0, an1 = lax.fori_loop(
                    0, cl, dist_body, (z, z, z), unroll=4
                )
                accp_v[c, ...] = ap
                accn0_v[c, ...] = an0
                accn1_v[c, ...] = an1

            zz = jnp.zeros((_LANES,), jnp.float32)
            dpv, dn0v, dn1v = zz, zz, zz
            for l in range(_LANES):
                coli = jnp.full((_LANES,), l, jnp.int32)
                dpv = dpv + plsc.load_gather(accp_v, [rowi, coli])
                dn0v = dn0v + plsc.load_gather(accn0_v, [rowi, coli])
                dn1v = dn1v + plsc.load_gather(accn1_v, [rowi, coli])
            return (
                jnp.maximum(dpv - dn0v + _MARGIN, 0.0) * w0vec
                + jnp.maximum(dpv - dn1v + _MARGIN, 0.0) * w1vec
            )

        stage_and_issue(0, sets[0])

        def chunk_pair(g2, tot):
            c0 = pl.multiple_of(g2 * (2 * _CH), _CH)
            c1 = pl.multiple_of(c0 + _CH, _CH)
            stage_and_issue(c1, sets[1])
            drain(sets[0])
            tot = tot + compute(c0, sets[0])

            @pl.when(g2 < n2 - 1)
            def _():
                stage_and_issue(c1 + _CH, sets[0])

            drain(sets[1])
            tot = tot + compute(c1, sets[1])
            return tot

        tot = lax.fori_loop(0, n2, chunk_pair, jnp.zeros((_LANES,), jnp.float32))
        out_v[...] = tot
        pltpu.sync_copy(out_v, out_hbm.at[wid])

    return pl.kernel(
        body,
        out_type=jax.ShapeDtypeStruct((n_workers, _LANES), jnp.float32),
        mesh=plsc.VectorSubcoreMesh(core_axis_name="c", subcore_axis_name="s"),
        compiler_params=pltpu.CompilerParams(needs_layout_passes=False),
        scratch_types=[
            pltpu.VMEM((4, per_w), jnp.int32),
            pltpu.VMEM((2, per_w), jnp.float32),
            pltpu.VMEM((2, per_w), jnp.float32),
            pltpu.VMEM((_CH,), jnp.int32),
            pltpu.VMEM((_CH,), jnp.int32),
            pltpu.VMEM((_CH,), jnp.int32),
            pltpu.VMEM((_CH,), jnp.int32),
            pltpu.VMEM((_CH,), jnp.int32),
            pltpu.VMEM((_CH,), jnp.int32),
            pltpu.VMEM((_CH,), jnp.int32),
            pltpu.VMEM((_CH,), jnp.int32),
            pltpu.VMEM((_CH, C), jnp.float32),
            pltpu.VMEM((_CH, C), jnp.float32),
            pltpu.VMEM((_CH, C), jnp.float32),
            pltpu.VMEM((_CH, C), jnp.float32),
            pltpu.VMEM((_CH, C), jnp.float32),
            pltpu.VMEM((_CH, C), jnp.float32),
            pltpu.VMEM((_CH, C), jnp.float32),
            pltpu.VMEM((_CH, C), jnp.float32),
            pltpu.VMEM((_CH, _LANES), jnp.float32),
            pltpu.VMEM((_CH, _LANES), jnp.float32),
            pltpu.VMEM((_CH, _LANES), jnp.float32),
            pltpu.VMEM((_LANES,), jnp.float32),
            pltpu.SemaphoreType.DMA,
            pltpu.SemaphoreType.DMA,
        ],
    )


def kernel(sketch_context_vectors, ref_context_vectors, G):
    B, H, W, _ = G.shape
    _, C, Hf, Wf = sketch_context_vectors.shape
    info = plsc.get_sparse_core_info()
    n_cores, n_subcores = info.num_cores, info.num_subcores
    n_workers = n_cores * n_subcores

    idx_slab, w_slab, bs_p, hs_p, ws_p, M, M_pad = _build_tables(
        int(B), int(H), int(W), n_workers
    )
    per_w = M_pad // n_workers

    sk_rows = jnp.transpose(sketch_context_vectors, (0, 2, 3, 1)).reshape(
        B * Hf * Wf, C
    )
    ref_rows = jnp.transpose(ref_context_vectors, (0, 2, 3, 1)).reshape(
        B * Hf * Wf, C
    )
    gxy = G[bs_p, hs_p * _RF, ws_p * _RF, :]
    g_slab = jnp.transpose(gxy.reshape(n_workers, per_w, 2), (0, 2, 1))

    fn = _sc_kernel(n_workers, n_cores, per_w, int(C), int(Wf))
    partials = fn(
        sk_rows,
        ref_rows,
        jnp.asarray(idx_slab),
        jnp.asarray(w_slab),
        g_slab,
    )
    return jnp.sum(partials)
